# Initial kernel scaffold; baseline (speedup 1.0000x reference)
import jax
import jax.numpy as jnp
from jax.experimental import pallas as pl


def kernel(Q, K, V, mask):
    raise NotImplementedError("write your pallas kernel here")


# speedup vs baseline: 1.0369x; 1.0369x over previous
import jax
import jax.numpy as jnp
from jax.experimental import pallas as pl
from jax.experimental.pallas import tpu as pltpu

QBLK = 512


def _attn_kernel(q_ref, k_ref, v_ref, o_ref, sks_ref, stv_ref):
    j = pl.program_id(1)
    n, d = k_ref.shape[1], k_ref.shape[2]
    m2 = sks_ref.shape[0]
    pool = n // m2

    @pl.when(j == 0)
    def _pool():
        sks_ref[...] = jnp.mean(k_ref[0].reshape(m2, pool, d), axis=1)
        stv_ref[...] = jnp.mean(v_ref[0].reshape(m2, pool, d), axis=1)

    q = q_ref[0]
    s = jax.lax.dot_general(
        q, sks_ref[...], (((1,), (1,)), ((), ())),
        preferred_element_type=jnp.float32) * (1.0 / (d ** 0.5))
    m = jnp.max(s, axis=-1, keepdims=True)
    e = jnp.exp(s - m)
    p = e / jnp.sum(e, axis=-1, keepdims=True)
    vres = v_ref[0, pl.ds(j * QBLK, QBLK), :]
    o_ref[0] = jax.lax.dot_general(
        p, stv_ref[...], (((1,), (0,)), ((), ())),
        preferred_element_type=jnp.float32) + vres


def kernel(Q, K, V, mask):
    b, h, n, d = Q.shape
    m2 = 256
    bh = b * h
    nq = n // QBLK
    Qf = Q.reshape(bh, n, d)
    Kf = K.reshape(bh, n, d)
    Vf = V.reshape(bh, n, d)
    out = pl.pallas_call(
        _attn_kernel,
        grid=(bh, nq),
        in_specs=[
            pl.BlockSpec((1, QBLK, d), lambda i, j: (i, j, 0)),
            pl.BlockSpec((1, n, d), lambda i, j: (i, 0, 0)),
            pl.BlockSpec((1, n, d), lambda i, j: (i, 0, 0)),
        ],
        out_specs=pl.BlockSpec((1, QBLK, d), lambda i, j: (i, j, 0)),
        out_shape=jax.ShapeDtypeStruct((bh, n, d), jnp.float32),
        scratch_shapes=[
            pltpu.VMEM((m2, d), jnp.float32),
            pltpu.VMEM((m2, d), jnp.float32),
        ],
        compiler_params=pltpu.CompilerParams(
            dimension_semantics=("arbitrary", "arbitrary")),
    )(Qf, Kf, Vf)
    return out.reshape(b, h, n, d)
